# baseline (speedup 1.0000x reference)
import functools

import jax
import jax.numpy as jnp
from jax import lax
from jax.experimental import pallas as pl
from jax.experimental.pallas import tpu as pltpu
from jax.experimental.pallas import tpu_sc as plsc

NC = 2
NS = 16
NW = NC * NS

B = 4
T = 2048
D = 768
LANES = 16
DV = D // LANES

TW = T // NW
CH = 16
PERB = TW // CH
NCHUNK = B * PERB
NBUF = 3


def _emb_body(
    idx_hbm, pos_hbm, tok_hbm, out_hbm,
    idx_v, pos_v, rows_v,
    isem, gsem0, gsem1, wsem0, wsem1, psem,
):
    wid = lax.axis_index("s") * NC + lax.axis_index("c")
    t0 = wid * TW
    gsems = (gsem0, gsem1)
    wsems = (wsem0, wsem1)

    def row0(h):
        return (h // PERB) * T + t0 + (h % PERB) * CH

    i0 = pltpu.async_copy(
        idx_hbm.at[pl.ds(row0(0), CH)], idx_v.at[pl.ds(0, CH)], isem
    )

    def gathers(h):
        return [
            pltpu.async_copy(
                tok_hbm.at[idx_v.at[pl.ds(h * CH, CH)]],
                rows_v.at[h % NBUF],
                gsems[h % 2],
            )
        ]

    i0.wait()
    g = {0: gathers(0)}
    pq = [
        pltpu.async_copy(
            pos_hbm.at[pl.ds(t0 + q * CH, CH)], pos_v.at[q], psem
        )
        for q in range(PERB)
    ]
    irest = [
        pltpu.async_copy(
            idx_hbm.at[pl.ds(b * T + t0 + (CH if b == 0 else 0), TW - (CH if b == 0 else 0))],
            idx_v.at[pl.ds(b * TW + (CH if b == 0 else 0), TW - (CH if b == 0 else 0))],
            isem,
        )
        for b in range(B)
    ]
    for cp in irest:
        cp.wait()
    for cp in pq:
        cp.wait()

    w = {}
    for h in range(NCHUNK):
        if h + 1 < NCHUNK:
            prev = h + 1 - NBUF
            if prev >= 0:
                for cp in w[prev]:
                    cp.wait()
            g[h + 1] = gathers(h + 1)
        for cp in g[h]:
            cp.wait()

        buf = rows_v.at[h % NBUF]
        q = h % PERB

        def j_body(j, _):
            sl = pl.ds(j * LANES, LANES)
            for r in range(CH):
                plsc.addupdate(buf.at[r, sl], pos_v[q, r, sl])
            return _

        lax.fori_loop(0, DV, j_body, 0)

        w[h] = [
            pltpu.async_copy(
                buf, out_hbm.at[pl.ds(row0(h), CH)], wsems[h % 2]
            )
        ]
    for h in range(max(0, NCHUNK - NBUF + 1), NCHUNK):
        for cp in w[h]:
            cp.wait()


@functools.lru_cache(maxsize=None)
def _emb_call():
    return functools.partial(
        pl.kernel,
        out_type=jax.ShapeDtypeStruct((B * T, D), jnp.float32),
        mesh=plsc.VectorSubcoreMesh(
            core_axis_name="c", subcore_axis_name="s", num_cores=NC, num_subcores=NS
        ),
        scratch_types=[
            pltpu.VMEM((B * TW,), jnp.int32),
            pltpu.VMEM((PERB, CH, D), jnp.float32),
            pltpu.VMEM((NBUF, CH, D), jnp.float32),
            pltpu.SemaphoreType.DMA,
            pltpu.SemaphoreType.DMA,
            pltpu.SemaphoreType.DMA,
            pltpu.SemaphoreType.DMA,
            pltpu.SemaphoreType.DMA,
            pltpu.SemaphoreType.DMA,
        ],
    )(_emb_body)


@jax.jit
def kernel(idx, tok_emb, pos_emb):
    b, t = idx.shape
    idx_flat = idx.reshape(b * t).astype(jnp.int32)
    pos2d = pos_emb.reshape(pos_emb.shape[1], pos_emb.shape[2])[:t]
    out = _emb_call()(idx_flat, pos2d, tok_emb)
    return out.reshape(b, t, pos_emb.shape[2])


# speedup vs baseline: 1.0838x; 1.0838x over previous
import functools

import jax
import jax.numpy as jnp
from jax import lax
from jax.experimental import pallas as pl
from jax.experimental.pallas import tpu as pltpu
from jax.experimental.pallas import tpu_sc as plsc

NC = 2
NS = 16
NW = NC * NS

B = 4
T = 2048
D = 768
LANES = 16
DV = D // LANES

TW = T // NW
CH = 32
PERB = TW // CH
NCHUNK = B * PERB
NBUF = 3


def _emb_body(
    idx_hbm, pos_hbm, tok_hbm, out_hbm,
    idx_v, pos_v, rows_v,
    isem, gsem0, gsem1, wsem0, wsem1, psem,
):
    wid = lax.axis_index("s") * NC + lax.axis_index("c")
    t0 = wid * TW
    gsems = (gsem0, gsem1)
    wsems = (wsem0, wsem1)

    def row0(h):
        return (h // PERB) * T + t0 + (h % PERB) * CH

    i0 = pltpu.async_copy(
        idx_hbm.at[pl.ds(row0(0), CH)], idx_v.at[pl.ds(0, CH)], isem
    )

    def gathers(h):
        return [
            pltpu.async_copy(
                tok_hbm.at[idx_v.at[pl.ds(h * CH, CH)]],
                rows_v.at[h % NBUF],
                gsems[h % 2],
            )
        ]

    i0.wait()
    g = {0: gathers(0)}
    pq = [
        pltpu.async_copy(
            pos_hbm.at[pl.ds(t0 + q * CH, CH)], pos_v.at[q], psem
        )
        for q in range(PERB)
    ]
    irest = [
        pltpu.async_copy(
            idx_hbm.at[pl.ds(b * T + t0 + (CH if b == 0 else 0), TW - (CH if b == 0 else 0))],
            idx_v.at[pl.ds(b * TW + (CH if b == 0 else 0), TW - (CH if b == 0 else 0))],
            isem,
        )
        for b in range(B)
    ]
    for cp in irest:
        cp.wait()
    for cp in pq:
        cp.wait()

    w = {}
    for h in range(NCHUNK):
        if h + 1 < NCHUNK:
            prev = h + 1 - NBUF
            if prev >= 0:
                for cp in w[prev]:
                    cp.wait()
            g[h + 1] = gathers(h + 1)
        for cp in g[h]:
            cp.wait()

        buf = rows_v.at[h % NBUF]
        q = h % PERB

        def j_body(j, _):
            sl = pl.ds(j * LANES, LANES)
            for r in range(CH):
                plsc.addupdate(buf.at[r, sl], pos_v[q, r, sl])
            return _

        lax.fori_loop(0, DV, j_body, 0)

        w[h] = [
            pltpu.async_copy(
                buf, out_hbm.at[pl.ds(row0(h), CH)], wsems[h % 2]
            )
        ]
    for h in range(max(0, NCHUNK - NBUF + 1), NCHUNK):
        for cp in w[h]:
            cp.wait()


@functools.lru_cache(maxsize=None)
def _emb_call():
    return functools.partial(
        pl.kernel,
        out_type=jax.ShapeDtypeStruct((B * T, D), jnp.float32),
        mesh=plsc.VectorSubcoreMesh(
            core_axis_name="c", subcore_axis_name="s", num_cores=NC, num_subcores=NS
        ),
        scratch_types=[
            pltpu.VMEM((B * TW,), jnp.int32),
            pltpu.VMEM((PERB, CH, D), jnp.float32),
            pltpu.VMEM((NBUF, CH, D), jnp.float32),
            pltpu.SemaphoreType.DMA,
            pltpu.SemaphoreType.DMA,
            pltpu.SemaphoreType.DMA,
            pltpu.SemaphoreType.DMA,
            pltpu.SemaphoreType.DMA,
            pltpu.SemaphoreType.DMA,
        ],
    )(_emb_body)


@jax.jit
def kernel(idx, tok_emb, pos_emb):
    b, t = idx.shape
    idx_flat = idx.reshape(b * t).astype(jnp.int32)
    pos2d = pos_emb.reshape(pos_emb.shape[1], pos_emb.shape[2])[:t]
    out = _emb_call()(idx_flat, pos2d, tok_emb)
    return out.reshape(b, t, pos_emb.shape[2])
